# Initial kernel scaffold; baseline (speedup 1.0000x reference)
import jax
import jax.numpy as jnp
from jax.experimental import pallas as pl


def kernel(hidden_states, weight):
    raise NotImplementedError("write your pallas kernel here")


# speedup vs baseline: 1.0644x; 1.0644x over previous
import jax
import jax.numpy as jnp
from jax.experimental import pallas as pl

TOP_K = 8
NUM_EXPERTS = 64
HIDDEN_DIM = 4096


def _router_body(x_ref, wt_ref, probs_ref, scores_ref, idx_ref):
    x = x_ref[...]
    wt = wt_ref[...]
    logits = jax.lax.dot_general(
        x, wt, (((1,), (0,)), ((), ())), preferred_element_type=jnp.float32
    )
    m = jnp.max(logits, axis=-1, keepdims=True)
    e = jnp.exp(logits - m)
    s = jnp.sum(e, axis=-1, keepdims=True)
    probs = e / s
    probs_ref[...] = probs

    work = probs
    iota = jax.lax.broadcasted_iota(jnp.int32, probs.shape, 1)
    vals = []
    idxs = []
    for _ in range(TOP_K):
        mj = jnp.max(work, axis=-1, keepdims=True)
        amj = jnp.min(
            jnp.where(work == mj, iota, NUM_EXPERTS), axis=-1, keepdims=True
        )
        vals.append(mj)
        idxs.append(amj)
        work = jnp.where(iota == amj, -1.0, work)
    v = jnp.concatenate(vals, axis=1)
    i = jnp.concatenate(idxs, axis=1)
    scores_ref[...] = v / jnp.sum(v, axis=1, keepdims=True)
    idx_ref[...] = i


def kernel(hidden_states, weight):
    x = hidden_states.reshape(-1, HIDDEN_DIM)
    wt = weight.T
    n_tokens = x.shape[0]
    bm = 512
    grid = (n_tokens // bm,)
    probs, scores, idx = pl.pallas_call(
        _router_body,
        grid=grid,
        in_specs=[
            pl.BlockSpec((bm, HIDDEN_DIM), lambda i: (i, 0)),
            pl.BlockSpec((HIDDEN_DIM, NUM_EXPERTS), lambda i: (0, 0)),
        ],
        out_specs=[
            pl.BlockSpec((bm, NUM_EXPERTS), lambda i: (i, 0)),
            pl.BlockSpec((bm, TOP_K), lambda i: (i, 0)),
            pl.BlockSpec((bm, TOP_K), lambda i: (i, 0)),
        ],
        out_shape=[
            jax.ShapeDtypeStruct((n_tokens, NUM_EXPERTS), jnp.float32),
            jax.ShapeDtypeStruct((n_tokens, TOP_K), jnp.float32),
            jax.ShapeDtypeStruct((n_tokens, TOP_K), jnp.int32),
        ],
    )(x, wt)
    return probs, scores, idx


# speedup vs baseline: 1.1784x; 1.1071x over previous
import jax
import jax.numpy as jnp
from jax.experimental import pallas as pl

TOP_K = 8
NUM_EXPERTS = 64
HIDDEN_DIM = 4096


def _router_body(x_ref, wt_ref, probs_ref, scores_ref, idx_ref):
    x = x_ref[...]
    wt = wt_ref[...]
    logits = jax.lax.dot_general(
        x, wt, (((1,), (0,)), ((), ())), preferred_element_type=jnp.float32
    )
    m = jnp.max(logits, axis=-1, keepdims=True)
    e = jnp.exp(logits - m)
    s = jnp.sum(e, axis=-1, keepdims=True)
    probs = e / s
    probs_ref[...] = probs

    work = probs
    iota = jax.lax.broadcasted_iota(jnp.int32, probs.shape, 1)
    vals = []
    idxs = []
    for _ in range(TOP_K):
        mj = jnp.max(work, axis=-1, keepdims=True)
        amj = jnp.min(
            jnp.where(work == mj, iota, NUM_EXPERTS), axis=-1, keepdims=True
        )
        vals.append(mj)
        idxs.append(amj)
        work = jnp.where(iota == amj, -1.0, work)
    v = jnp.concatenate(vals, axis=1)
    i = jnp.concatenate(idxs, axis=1)
    scores_ref[...] = v / jnp.sum(v, axis=1, keepdims=True)
    idx_ref[...] = i


def kernel(hidden_states, weight):
    x = hidden_states.reshape(-1, HIDDEN_DIM)
    wt = weight.T
    n_tokens = x.shape[0]
    bm = 1024
    grid = (n_tokens // bm,)
    probs, scores, idx = pl.pallas_call(
        _router_body,
        grid=grid,
        in_specs=[
            pl.BlockSpec((bm, HIDDEN_DIM), lambda i: (i, 0)),
            pl.BlockSpec((HIDDEN_DIM, NUM_EXPERTS), lambda i: (0, 0)),
        ],
        out_specs=[
            pl.BlockSpec((bm, NUM_EXPERTS), lambda i: (i, 0)),
            pl.BlockSpec((bm, TOP_K), lambda i: (i, 0)),
            pl.BlockSpec((bm, TOP_K), lambda i: (i, 0)),
        ],
        out_shape=[
            jax.ShapeDtypeStruct((n_tokens, NUM_EXPERTS), jnp.float32),
            jax.ShapeDtypeStruct((n_tokens, TOP_K), jnp.float32),
            jax.ShapeDtypeStruct((n_tokens, TOP_K), jnp.int32),
        ],
    )(x, wt)
    return probs, scores, idx
